# baseline (speedup 1.0000x reference)
import functools

import jax
import jax.numpy as jnp
from jax import lax
from jax.experimental import pallas as pl
from jax.experimental.pallas import tpu as pltpu
from jax.experimental.pallas import tpu_sc as plsc

N = 100000
D = 128
G = 512

NC = 2
NS = 16
HALF = D // NC
R = 512
NFULL = N // R
TAIL = N - NFULL * R
NB = NFULL // NS
EXTRA = NFULL - NB * NS
SEG_PER_TILE = G // NS
CH = R // 128
NJ = HALF // 16

_mesh = plsc.VectorSubcoreMesh(core_axis_name="c", subcore_axis_name="s")


@functools.partial(
    pl.kernel,
    out_type=jax.ShapeDtypeStruct((G, D), jnp.float32),
    mesh=_mesh,
    compiler_params=pltpu.CompilerParams(use_tc_tiling_on_sc=False,
                                         needs_layout_passes=False),
    scratch_types=[
        pltpu.VMEM((2, R, HALF), jnp.float32),
        pltpu.VMEM((2, CH, 128), jnp.int32),
        pltpu.VMEM((32,), jnp.int32),
        pltpu.VMEM((128, 16), jnp.float32),
        pltpu.VMEM((CH, 128), jnp.int32),
        pltpu.VMEM((G, HALF), jnp.float32),
        pltpu.VMEM((G, 16), jnp.float32),
        pltpu.VMEM((SEG_PER_TILE, HALF), jnp.float32),
        pltpu.VMEM((SEG_PER_TILE, 16), jnp.float32),
        pltpu.VMEM_SHARED((G, HALF), jnp.float32),
        pltpu.VMEM_SHARED((G, 16), jnp.float32),
        pltpu.SemaphoreType.DMA,
        pltpu.SemaphoreType.DMA,
        pltpu.SemaphoreType.DMA,
        pltpu.SemaphoreType.DMA,
        pltpu.SemaphoreType.DMA,
    ],
)
def _pool_kernel(x_hbm, b_hbm, z_hbm, ones_hbm, iota_hbm, out_hbm,
                 xb, idxb, idx_t, onesv, iotav, accl, cntl, accv, cntv,
                 acc_sh, cnt_sh, sx0, sx1, si0, si1, ssc):
    c = lax.axis_index("c")
    s = lax.axis_index("s")
    col0 = c * HALF
    seg0 = s * SEG_PER_TILE
    sx = (sx0, sx1)
    si = (si0, si1)
    zvec = jnp.zeros((16,), jnp.float32)
    ones16 = jnp.ones((16,), jnp.float32)
    full128 = jnp.full((16,), 128.0, jnp.float32)

    def start_load(kblk, b):
        r0 = (s + NS * kblk) * R
        pltpu.async_copy(x_hbm.at[pl.ds(r0, R), pl.ds(col0, HALF)],
                         xb.at[b], sx[b])
        for m in range(CH):
            pltpu.async_copy(b_hbm.at[pl.ds(r0 + 128 * m, 128)],
                             idxb.at[b, m], si[b])

    def wait_load(b):
        pltpu.make_async_copy(x_hbm.at[pl.ds(0, R), pl.ds(0, HALF)],
                              xb.at[b], sx[b]).wait()
        for m in range(CH):
            pltpu.make_async_copy(b_hbm.at[pl.ds(0, 128)],
                                  idxb.at[b, m], si[b]).wait()

    def process_chunk(b, m):
        ivs = [idxb[b, m, pl.ds(16 * u, 16)] for u in range(8)]
        sid0 = ivs[0][0]
        ok = jnp.all(ivs[0] == sid0)
        for u in range(1, 8):
            ok = jnp.logical_and(ok, jnp.all(ivs[u] == sid0))

        @pl.when(ok)
        def _uniform():
            plsc.addupdate(cntl.at[sid0], full128)

            def body(u, carry):
                accs = list(carry)
                for r in range(16):
                    row = 128 * m + 16 * u + r
                    for j in range(NJ):
                        accs[j] = accs[j] + xb[b, row, pl.ds(16 * j, 16)]
                return tuple(accs)

            a = lax.fori_loop(0, 8, body, (zvec,) * NJ)
            for j in range(NJ):
                plsc.addupdate(accl.at[sid0, pl.ds(16 * j, 16)], a[j])

        @pl.when(jnp.logical_not(ok))
        def _stream():
            pltpu.sync_copy(xb.at[b, pl.ds(128 * m, 128)],
                            acc_sh.at[idxb.at[b, m]], add=True)
            pltpu.sync_copy(onesv, cnt_sh.at[idxb.at[b, m]], add=True)

    def process_buf(b):
        for m in range(CH):
            process_chunk(b, m)

    start_load(0, 0)
    start_load(1, 1)
    pltpu.sync_copy(z_hbm, acc_sh.at[pl.ds(seg0, SEG_PER_TILE)])
    pltpu.sync_copy(z_hbm.at[:, pl.ds(0, 16)],
                    cnt_sh.at[pl.ds(seg0, SEG_PER_TILE)])
    pltpu.sync_copy(ones_hbm, onesv)
    pltpu.sync_copy(iota_hbm, iotav)

    def zrow(i, carry):
        for j in range(NJ):
            accl[i, pl.ds(16 * j, 16)] = zvec
        cntl[i, pl.ds(0, 16)] = zvec
        return carry

    lax.fori_loop(0, G, zrow, 0)
    plsc.subcore_barrier()

    def blk_body(g, carry):
        for b in range(2):
            wait_load(b)
            process_buf(b)
            start_load(2 * g + b + 2, b)
        return carry

    lax.fori_loop(0, NB // 2 - 1, blk_body, 0)
    for b in range(2):
        wait_load(b)
        process_buf(b)

    @pl.when(s < EXTRA)
    def _extra():
        r0 = (NB * NS + s) * R
        pltpu.sync_copy(x_hbm.at[pl.ds(r0, R), pl.ds(col0, HALF)], xb.at[0])
        for m in range(CH):
            pltpu.sync_copy(b_hbm.at[pl.ds(r0 + 128 * m, 128)], idxb.at[0, m])
        process_buf(0)

    @pl.when(s == EXTRA)
    def _tail():
        r0 = NFULL * R
        pltpu.sync_copy(x_hbm.at[pl.ds(r0, TAIL), pl.ds(col0, HALF)],
                        xb.at[0, pl.ds(0, TAIL)])
        pltpu.sync_copy(b_hbm.at[pl.ds(r0, 128)], idxb.at[0, 0])
        pltpu.sync_copy(b_hbm.at[pl.ds(r0 + 128, 32)], idx_t)
        process_chunk(0, 0)

        def tgrp(u, carry):
            idvec = idx_t[pl.ds(16 * u, 16)]
            for r in range(16):
                sid = idvec[r]
                plsc.addupdate(cntl.at[sid], ones16)
                for j in range(NJ):
                    xv = xb[0, 128 + 16 * u + r, pl.ds(16 * j, 16)]
                    plsc.addupdate(accl.at[sid, pl.ds(16 * j, 16)], xv)
            return carry

        lax.fori_loop(0, 2, tgrp, 0)

    handles = [pltpu.async_copy(accl.at[pl.ds(128 * q, 128)],
                                acc_sh.at[iotav.at[q]], ssc, add=True)
               for q in range(CH)]
    handles += [pltpu.async_copy(cntl.at[pl.ds(128 * q, 128)],
                                 cnt_sh.at[iotav.at[q]], ssc, add=True)
                for q in range(CH)]
    for h in handles:
        h.wait()
    plsc.subcore_barrier()

    pltpu.sync_copy(acc_sh.at[pl.ds(seg0, SEG_PER_TILE)], accv)
    pltpu.sync_copy(cnt_sh.at[pl.ds(seg0, SEG_PER_TILE)], cntv)
    for i in range(SEG_PER_TILE):
        inv = 1.0 / jnp.maximum(cntv[i, :], 1.0)
        for j in range(NJ):
            accv[i, pl.ds(16 * j, 16)] = accv[i, pl.ds(16 * j, 16)] * inv
    pltpu.sync_copy(accv,
                    out_hbm.at[pl.ds(seg0, SEG_PER_TILE), pl.ds(col0, HALF)])


def kernel(x, batch):
    zeros = jnp.zeros((SEG_PER_TILE, HALF), jnp.float32)
    ones = jnp.ones((128, 16), jnp.float32)
    iota = jnp.arange(G, dtype=jnp.int32).reshape(CH, 128)
    return _pool_kernel(x, batch, zeros, ones, iota)


# speedup vs baseline: 1.0306x; 1.0306x over previous
import functools

import jax
import jax.numpy as jnp
from jax import lax
from jax.experimental import pallas as pl
from jax.experimental.pallas import tpu as pltpu
from jax.experimental.pallas import tpu_sc as plsc

N = 100000
D = 128
G = 512

NC = 2
NS = 16
HALF = D // NC
R = 512
NFULL = N // R
TAIL = N - NFULL * R
NB = NFULL // NS
EXTRA = NFULL - NB * NS
SEG_PER_TILE = G // NS
CH = R // 128
NJ = HALF // 16

_mesh = plsc.VectorSubcoreMesh(core_axis_name="c", subcore_axis_name="s")


@functools.partial(
    pl.kernel,
    out_type=jax.ShapeDtypeStruct((G, D), jnp.float32),
    mesh=_mesh,
    compiler_params=pltpu.CompilerParams(use_tc_tiling_on_sc=False,
                                         needs_layout_passes=False),
    scratch_types=[
        pltpu.VMEM((2, R, HALF), jnp.float32),
        pltpu.VMEM((2, CH, 128), jnp.int32),
        pltpu.VMEM((32,), jnp.int32),
        pltpu.VMEM((128, 16), jnp.float32),
        pltpu.VMEM((CH, 128), jnp.int32),
        pltpu.VMEM((G, HALF), jnp.float32),
        pltpu.VMEM((G, 16), jnp.float32),
        pltpu.VMEM((SEG_PER_TILE, HALF), jnp.float32),
        pltpu.VMEM((SEG_PER_TILE, 16), jnp.float32),
        pltpu.VMEM_SHARED((G, HALF), jnp.float32),
        pltpu.VMEM_SHARED((G, 16), jnp.float32),
        pltpu.SemaphoreType.DMA,
        pltpu.SemaphoreType.DMA,
        pltpu.SemaphoreType.DMA,
        pltpu.SemaphoreType.DMA,
        pltpu.SemaphoreType.DMA,
    ],
)
def _pool_kernel(x_hbm, b_hbm, z_hbm, ones_hbm, iota_hbm, out_hbm,
                 xb, idxb, idx_t, onesv, iotav, accl, cntl, accv, cntv,
                 acc_sh, cnt_sh, sx0, sx1, si0, si1, ssc):
    c = lax.axis_index("c")
    s = lax.axis_index("s")
    col0 = c * HALF
    seg0 = s * SEG_PER_TILE
    sx = (sx0, sx1)
    si = (si0, si1)
    zvec = jnp.zeros((16,), jnp.float32)
    ones16 = jnp.ones((16,), jnp.float32)
    full128 = jnp.full((16,), 128.0, jnp.float32)

    def start_load(kblk, b):
        r0 = (s + NS * kblk) * R
        pltpu.async_copy(x_hbm.at[pl.ds(r0, R), pl.ds(col0, HALF)],
                         xb.at[b], sx[b])
        for m in range(CH):
            pltpu.async_copy(b_hbm.at[pl.ds(r0 + 128 * m, 128)],
                             idxb.at[b, m], si[b])

    def wait_load(b):
        pltpu.make_async_copy(x_hbm.at[pl.ds(0, R), pl.ds(0, HALF)],
                              xb.at[b], sx[b]).wait()
        for m in range(CH):
            pltpu.make_async_copy(b_hbm.at[pl.ds(0, 128)],
                                  idxb.at[b, m], si[b]).wait()

    def chunk_uniform(b, m):
        ivs = [idxb[b, m, pl.ds(16 * u, 16)] for u in range(8)]
        sid0 = ivs[0][0]
        ok = jnp.all(ivs[0] == sid0)
        for u in range(1, 8):
            ok = jnp.logical_and(ok, jnp.all(ivs[u] == sid0))
        return ok, sid0

    def tree_reduce_chunk(b, m, sid0):
        plsc.addupdate(cntl.at[sid0], full128)

        def body(u, carry):
            accs = list(carry)
            for r in range(16):
                row = 128 * m + 16 * u + r
                for j in range(NJ):
                    accs[j] = accs[j] + xb[b, row, pl.ds(16 * j, 16)]
            return tuple(accs)

        a = lax.fori_loop(0, 8, body, (zvec,) * NJ)
        for j in range(NJ):
            plsc.addupdate(accl.at[sid0, pl.ds(16 * j, 16)], a[j])

    def process_buf(b):
        oks = []
        sids = []
        for m in range(CH):
            ok, sid0 = chunk_uniform(b, m)
            oks.append(ok)
            sids.append(sid0)

            @pl.when(jnp.logical_not(ok))
            def _fire(b=b, m=m):
                pltpu.async_copy(xb.at[b, pl.ds(128 * m, 128)],
                                 acc_sh.at[idxb.at[b, m]], ssc, add=True)
                pltpu.async_copy(onesv, cnt_sh.at[idxb.at[b, m]], ssc,
                                 add=True)

        for m in range(CH):
            @pl.when(oks[m])
            def _tree(b=b, m=m, sid0=sids[m]):
                tree_reduce_chunk(b, m, sid0)

        for m in range(CH):
            @pl.when(jnp.logical_not(oks[m]))
            def _drain(b=b, m=m):
                pltpu.make_async_copy(xb.at[b, pl.ds(128 * m, 128)],
                                      acc_sh.at[idxb.at[b, m]], ssc).wait()
                pltpu.make_async_copy(onesv, cnt_sh.at[idxb.at[b, m]],
                                      ssc).wait()

    start_load(0, 0)
    start_load(1, 1)
    pltpu.sync_copy(z_hbm, acc_sh.at[pl.ds(seg0, SEG_PER_TILE)])
    pltpu.sync_copy(z_hbm.at[:, pl.ds(0, 16)],
                    cnt_sh.at[pl.ds(seg0, SEG_PER_TILE)])
    pltpu.sync_copy(ones_hbm, onesv)
    pltpu.sync_copy(iota_hbm, iotav)

    def zrow(i, carry):
        for j in range(NJ):
            accl[i, pl.ds(16 * j, 16)] = zvec
        cntl[i, pl.ds(0, 16)] = zvec
        return carry

    lax.fori_loop(0, G, zrow, 0)
    plsc.subcore_barrier()

    def blk_body(g, carry):
        for b in range(2):
            wait_load(b)
            process_buf(b)
            start_load(2 * g + b + 2, b)
        return carry

    lax.fori_loop(0, NB // 2 - 1, blk_body, 0)
    for b in range(2):
        wait_load(b)
        process_buf(b)

    @pl.when(s < EXTRA)
    def _extra():
        r0 = (NB * NS + s) * R
        pltpu.sync_copy(x_hbm.at[pl.ds(r0, R), pl.ds(col0, HALF)], xb.at[0])
        for m in range(CH):
            pltpu.sync_copy(b_hbm.at[pl.ds(r0 + 128 * m, 128)], idxb.at[0, m])
        process_buf(0)

    @pl.when(s == EXTRA)
    def _tail():
        r0 = NFULL * R
        pltpu.sync_copy(x_hbm.at[pl.ds(r0, TAIL), pl.ds(col0, HALF)],
                        xb.at[0, pl.ds(0, TAIL)])
        pltpu.sync_copy(b_hbm.at[pl.ds(r0, 128)], idxb.at[0, 0])
        pltpu.sync_copy(b_hbm.at[pl.ds(r0 + 128, 32)], idx_t)
        ok0, tsid0 = chunk_uniform(0, 0)

        @pl.when(ok0)
        def _ttree():
            tree_reduce_chunk(0, 0, tsid0)

        @pl.when(jnp.logical_not(ok0))
        def _tstream():
            pltpu.sync_copy(xb.at[0, pl.ds(0, 128)],
                            acc_sh.at[idxb.at[0, 0]], add=True)
            pltpu.sync_copy(onesv, cnt_sh.at[idxb.at[0, 0]], add=True)

        def tgrp(u, carry):
            idvec = idx_t[pl.ds(16 * u, 16)]
            for r in range(16):
                sid = idvec[r]
                plsc.addupdate(cntl.at[sid], ones16)
                for j in range(NJ):
                    xv = xb[0, 128 + 16 * u + r, pl.ds(16 * j, 16)]
                    plsc.addupdate(accl.at[sid, pl.ds(16 * j, 16)], xv)
            return carry

        lax.fori_loop(0, 2, tgrp, 0)

    handles = [pltpu.async_copy(accl.at[pl.ds(128 * q, 128)],
                                acc_sh.at[iotav.at[q]], ssc, add=True)
               for q in range(CH)]
    handles += [pltpu.async_copy(cntl.at[pl.ds(128 * q, 128)],
                                 cnt_sh.at[iotav.at[q]], ssc, add=True)
                for q in range(CH)]
    for h in handles:
        h.wait()
    plsc.subcore_barrier()

    pltpu.sync_copy(acc_sh.at[pl.ds(seg0, SEG_PER_TILE)], accv)
    pltpu.sync_copy(cnt_sh.at[pl.ds(seg0, SEG_PER_TILE)], cntv)
    for i in range(SEG_PER_TILE):
        inv = 1.0 / jnp.maximum(cntv[i, :], 1.0)
        for j in range(NJ):
            accv[i, pl.ds(16 * j, 16)] = accv[i, pl.ds(16 * j, 16)] * inv
    pltpu.sync_copy(accv,
                    out_hbm.at[pl.ds(seg0, SEG_PER_TILE), pl.ds(col0, HALF)])


def kernel(x, batch):
    zeros = jnp.zeros((SEG_PER_TILE, HALF), jnp.float32)
    ones = jnp.ones((128, 16), jnp.float32)
    iota = jnp.arange(G, dtype=jnp.int32).reshape(CH, 128)
    return _pool_kernel(x, batch, zeros, ones, iota)
